# Initial kernel scaffold; baseline (speedup 1.0000x reference)
import jax
import jax.numpy as jnp
from jax.experimental import pallas as pl


def kernel(obs, action, next_obs, enc_w1, enc_b1, enc_w2, enc_b2, enc_g, enc_beta, enc_w3, enc_b3, edge_w1, edge_b1, edge_w2, edge_b2, edge_g, edge_beta, edge_w3, edge_b3, node_w1, node_b1, node_w2, node_b2, node_g, node_beta, node_w3, node_b3):
    raise NotImplementedError("write your pallas kernel here")


# speedup vs baseline: 8.4347x; 8.4347x over previous
import functools

import numpy as np
import jax
import jax.numpy as jnp
from jax import lax
from jax.experimental import pallas as pl
from jax.experimental.pallas import tpu as pltpu
from jax.experimental.pallas import tpu_sc as plsc

B = 4096
O = 5
DIN = 25
HID = 128
EMB = 128
HINGE = 1.0
SIGMA = 0.5
NORM = 0.5 / (SIGMA ** 2)

BB = 512
NB = B // BB
R = O * BB

_PERM = np.random.RandomState(0).permutation(B)
_NPERM = (np.arange(O, dtype=np.int64)[:, None] * B
          + _PERM[None, :].astype(np.int64)).reshape(-1).astype(np.int32)


def _mm(a, b):
    return lax.dot_general(a, b, (((1,), (0,)), ((), ())),
                           preferred_element_type=jnp.float32)


def _ln(x, g, b):
    m = jnp.mean(x, axis=-1, keepdims=True)
    v = jnp.mean((x - m) ** 2, axis=-1, keepdims=True)
    return (x - m) * lax.rsqrt(v + 1e-5) * g + b


def _tc_body(obs_ref, nobs_ref, act_ref,
             ew1, eb1, ew2, eb2, eg, ebe, ew3, eb3,
             gw1a, gw1b, gb1, gw2, gb2, gg, gbe, gw3, gb3,
             nw1s, nw1v, nw1a, nb1, nw2, nb2, ng, nbe, nw3, nb3,
             state_out, pos_out):
    nb = pl.program_id(0)

    def encoder(x):
        h = jax.nn.relu(_mm(x, ew1[...]) + eb1[...])
        h = jax.nn.relu(_ln(_mm(h, ew2[...]) + eb2[...], eg[...], ebe[...]))
        return _mm(h, ew3[...]) + eb3[...]

    state = encoder(obs_ref[...].reshape(R, DIN))
    nstate = encoder(nobs_ref[...].reshape(R, DIN))

    u = _mm(state, gw1a[...])
    v = _mm(state, gw1b[...]) + gb1[...]
    agg = jnp.zeros((R, HID), jnp.float32)
    for s in range(1, O):
        vs = jnp.concatenate([v[s * BB:], v[:s * BB]], axis=0)
        e = jax.nn.relu(u + vs)
        e = jax.nn.relu(_ln(_mm(e, gw2[...]) + gb2[...], gg[...], gbe[...]))
        agg = agg + _mm(e, gw3[...]) + gb3[...]

    act = act_ref[...].reshape(R, 1)
    i_of_row = lax.broadcasted_iota(jnp.int32, (R, 1), 0) // BB
    ak = act - 4 * i_of_row
    h = _mm(state, nw1s[...]) + _mm(agg, nw1a[...]) + nb1[...]
    for k in range(4):
        h = h + (ak == k).astype(jnp.float32) * nw1v[k:k + 1, :]
    h = jax.nn.relu(h)
    h = jax.nn.relu(_ln(_mm(h, nw2[...]) + nb2[...], ng[...], nbe[...]))
    pred = _mm(h, nw3[...]) + nb3[...]

    d = state + pred - nstate
    rowsq = jnp.sum(d * d, axis=-1, keepdims=True)
    ps = rowsq[0:BB]
    for i in range(1, O):
        ps = ps + rowsq[i * BB:(i + 1) * BB]
    blk = (NORM / O) * jnp.sum(ps, keepdims=True).reshape(1, 1)

    state_out[...] = state.reshape(O, BB, EMB)

    @pl.when(nb == 0)
    def _():
        pos_out[...] = jnp.zeros((1, 1), jnp.float32)
    pos_out[...] += blk


def _tc_forward(obs_t, nobs_t, act3, weights, interpret=False):
    full = lambda w: pl.BlockSpec(w.shape, lambda nb, _n=None: (0,) * w.ndim)
    in_specs = [
        pl.BlockSpec((O, BB, DIN), lambda nb: (0, nb, 0)),
        pl.BlockSpec((O, BB, DIN), lambda nb: (0, nb, 0)),
        pl.BlockSpec((O, BB, 1), lambda nb: (0, nb, 0)),
    ] + [full(w) for w in weights]
    return pl.pallas_call(
        _tc_body,
        grid=(NB,),
        in_specs=in_specs,
        out_specs=[
            pl.BlockSpec((O, BB, EMB), lambda nb: (0, nb, 0)),
            pl.BlockSpec((1, 1), lambda nb: (0, 0)),
        ],
        out_shape=[
            jax.ShapeDtypeStruct((O, B, EMB), jnp.float32),
            jax.ShapeDtypeStruct((1, 1), jnp.float32),
        ],
        interpret=interpret,
    )(obs_t, nobs_t, act3, *weights)


_NW = 32
_CH = B // _NW


def _sc_neg(state2d, nperm):
    mesh = plsc.VectorSubcoreMesh(core_axis_name="c", subcore_axis_name="s")

    @functools.partial(
        pl.kernel,
        mesh=mesh,
        out_type=jax.ShapeDtypeStruct((B * 16,), jnp.float32),
        scratch_types=[
            pltpu.VMEM((_CH,), jnp.int32),
            pltpu.VMEM((_CH, EMB), jnp.float32),
            pltpu.VMEM((_CH, EMB), jnp.float32),
            pltpu.VMEM((_CH * 16,), jnp.float32),
            pltpu.SemaphoreType.DMA,
        ],
    )
    def k(state_hbm, nperm_hbm, out_hbm, idx_v, gat_v, lin_v, sums_v, sem):
        wid = lax.axis_index("s") * 2 + lax.axis_index("c")
        base = wid * _CH
        zz = jnp.zeros((16,), jnp.float32)

        def zero_body(r, _):
            sums_v[pl.ds(r * 16, 16)] = zz
            return 0

        lax.fori_loop(0, _CH, zero_body, 0)
        for i in range(O):
            pltpu.sync_copy(nperm_hbm.at[pl.ds(i * B + base, _CH)], idx_v)
            pltpu.async_copy(state_hbm.at[idx_v], gat_v, sem).wait()
            pltpu.sync_copy(state_hbm.at[pl.ds(i * B + base, _CH)], lin_v)

            def row_body(r, _):
                acc = jnp.zeros((16,), jnp.float32)
                for c in range(EMB // 16):
                    dd = (lin_v[r, pl.ds(c * 16, 16)]
                          - gat_v[r, pl.ds(c * 16, 16)])
                    acc = acc + dd * dd
                sl = pl.ds(r * 16, 16)
                sums_v[sl] = sums_v[sl] + acc
                return 0

            lax.fori_loop(0, _CH, row_body, 0)
        pltpu.sync_copy(sums_v, out_hbm.at[pl.ds(base * 16, _CH * 16)])

    return k(state2d, nperm)


def _tc_hinge_body(parts_ref, neg_out):
    p = parts_ref[...]
    s = jnp.sum(p, axis=-1, keepdims=True)
    h = jnp.maximum(0.0, HINGE - (NORM / O) * s)
    neg_out[...] = jnp.sum(h, keepdims=True).reshape(1, 1)


def _tc_hinge(parts):
    return pl.pallas_call(
        _tc_hinge_body,
        out_shape=jax.ShapeDtypeStruct((1, 1), jnp.float32),
    )(parts)


def kernel(obs, action, next_obs,
           enc_w1, enc_b1, enc_w2, enc_b2, enc_g, enc_beta, enc_w3, enc_b3,
           edge_w1, edge_b1, edge_w2, edge_b2, edge_g, edge_beta, edge_w3,
           edge_b3, node_w1, node_b1, node_w2, node_b2, node_g, node_beta,
           node_w3, node_b3):
    obs_t = obs.reshape(B, O, DIN).transpose(1, 0, 2)
    nobs_t = next_obs.reshape(B, O, DIN).transpose(1, 0, 2)
    act3 = jnp.broadcast_to(action.astype(jnp.int32)[None, :, None], (O, B, 1))
    row = lambda x: x.reshape(1, -1)
    weights = [
        enc_w1, row(enc_b1), enc_w2, row(enc_b2), row(enc_g), row(enc_beta),
        enc_w3, row(enc_b3),
        edge_w1[:EMB], edge_w1[EMB:], row(edge_b1), edge_w2, row(edge_b2),
        row(edge_g), row(edge_beta), edge_w3, row(edge_b3),
        node_w1[:EMB], node_w1[EMB:EMB + 4], node_w1[EMB + 4:], row(node_b1),
        node_w2, row(node_b2), row(node_g), row(node_beta), node_w3,
        row(node_b3),
    ]
    state3, pos_sum = _tc_forward(obs_t, nobs_t, act3, weights)
    neg_parts = _sc_neg(state3.reshape(O * B, EMB),
                        jnp.asarray(_NPERM))
    neg_sum = _tc_hinge(neg_parts.reshape(B, 16))
    return pos_sum[0, 0] / B + neg_sum[0, 0] / B
